# Initial kernel scaffold; baseline (speedup 1.0000x reference)
import jax
import jax.numpy as jnp
from jax.experimental import pallas as pl


def kernel(x, edge_index, W1, b1, W2, b2, Wl, bl):
    raise NotImplementedError("write your pallas kernel here")


# speedup vs baseline: 8.7027x; 8.7027x over previous
import functools

import jax
import jax.numpy as jnp
from jax import lax
from jax.experimental import pallas as pl
from jax.experimental.pallas import tpu as pltpu
from jax.experimental.pallas import tpu_sc as plsc

N = 10000
E = 320000
H = 128
NC = 2
NS = 16
NW = NC * NS
EPW = E // NW
CH = 80
NCHUNK = EPW // CH
RSTRIDE = 624
RLEN = 640

_mesh = plsc.VectorSubcoreMesh(core_axis_name="c", subcore_axis_name="s")


def _zero_fill(buf, nrows, ncols, value=0.0):
    vec = jnp.full((16,), value, dtype=jnp.float32)

    def body(i, _):
        for j in range(ncols // 16):
            buf[i, pl.ds(j * 16, 16)] = vec
        return 0

    lax.fori_loop(0, nrows, body, 0)


def _deg_body(dst_hbm, out_hbm, dacc, dst_idx, ones80, initbuf, gsem):
    c = lax.axis_index("c")
    s = lax.axis_index("s")
    wid = s * NC + c
    base = pl.multiple_of(s * RSTRIDE, 8)

    initval = jnp.where(c == 0, 1.0, 0.0)
    _zero_fill(initbuf, RLEN, 16, initval)
    _zero_fill(ones80, CH, 16, 1.0)
    pltpu.sync_copy(initbuf, dacc.at[pl.ds(base, RLEN)])
    plsc.subcore_barrier()

    def body(i, _):
        off = pl.multiple_of(wid * EPW + i * CH, 8)
        pltpu.sync_copy(dst_hbm.at[pl.ds(off, CH)], dst_idx)
        pltpu.sync_copy(ones80, dacc.at[dst_idx], add=True)
        return 0

    lax.fori_loop(0, NCHUNK, body, 0)
    plsc.subcore_barrier()
    pltpu.sync_copy(dacc.at[pl.ds(base, RLEN)], out_hbm.at[c, pl.ds(base, RLEN)])


_deg_partials = pl.kernel(
    _deg_body,
    out_type=jax.ShapeDtypeStruct((NC, N, 16), jnp.float32),
    mesh=_mesh,
    scratch_types=[
        pltpu.VMEM_SHARED((N, 16), jnp.float32),
        pltpu.VMEM((CH,), jnp.int32),
        pltpu.VMEM((CH, 16), jnp.float32),
        pltpu.VMEM((RLEN, 16), jnp.float32),
        pltpu.SemaphoreType.DMA,
    ],
    compiler_params=pltpu.CompilerParams(use_tc_tiling_on_sc=False),
)


HH = H // 2
EPT = E // NS
NCHUNK2 = EPT // CH


def _agg_body(y_hbm, src_hbm, dst_hbm, out_hbm, acc, src_idx, dst_idx, rows,
              gsem):
    c = lax.axis_index("c")
    s = lax.axis_index("s")
    base = pl.multiple_of(s * RSTRIDE, 8)

    pltpu.sync_copy(y_hbm.at[c, pl.ds(base, RLEN)], acc.at[pl.ds(base, RLEN)])
    plsc.subcore_barrier()

    def body(i, _):
        off = pl.multiple_of(s * EPT + i * CH, 8)
        pltpu.sync_copy(src_hbm.at[pl.ds(off, CH)], src_idx)
        pltpu.sync_copy(dst_hbm.at[pl.ds(off, CH)], dst_idx)
        pltpu.async_copy(y_hbm.at[c].at[src_idx], rows, gsem).wait()
        pltpu.sync_copy(rows, acc.at[dst_idx], add=True)
        return 0

    lax.fori_loop(0, NCHUNK2, body, 0)
    plsc.subcore_barrier()
    pltpu.sync_copy(acc.at[pl.ds(base, RLEN)], out_hbm.at[c, pl.ds(base, RLEN)])


_agg_halves = pl.kernel(
    _agg_body,
    out_type=jax.ShapeDtypeStruct((NC, N, HH), jnp.float32),
    mesh=_mesh,
    scratch_types=[
        pltpu.VMEM_SHARED((N, HH), jnp.float32),
        pltpu.VMEM((CH,), jnp.int32),
        pltpu.VMEM((CH,), jnp.int32),
        pltpu.VMEM((CH, HH), jnp.float32),
        pltpu.SemaphoreType.DMA,
    ],
    compiler_params=pltpu.CompilerParams(use_tc_tiling_on_sc=False),
)


_RB = 1000


def _dis_from(degp):
    deg = degp[0] + degp[1]
    return lax.rsqrt(jnp.maximum(deg[:, 0:1], 1.0))


def _y_body(flag_ref, acch_ref, degp_ref, x_ref, b_ref, w_ref, y_ref):
    dis = _dis_from(degp_ref)
    a = jnp.concatenate([acch_ref[0], acch_ref[1]], axis=1)
    h = jnp.maximum(a * dis + b_ref[...], 0.0)
    pre = jnp.where(flag_ref[...] > 0, h, x_ref[...])
    y = jnp.dot(pre, w_ref[...], preferred_element_type=jnp.float32) * dis
    y_ref[0] = y[:, :HH]
    y_ref[1] = y[:, HH:]


def _z_body(acch_ref, degp_ref, b2_ref, wl_ref, bl_ref, z_ref):
    dis = _dis_from(degp_ref)
    a = jnp.concatenate([acch_ref[0], acch_ref[1]], axis=1)
    h = jnp.maximum(a * dis + b2_ref[...], 0.0)
    z_ref[...] = jnp.dot(h, wl_ref[...],
                         preferred_element_type=jnp.float32) + bl_ref[...]


def _full(shape):
    return pl.BlockSpec(shape, lambda i: (0,) * len(shape))


_degp_spec = pl.BlockSpec((NC, _RB, 16), lambda i: (0, i, 0))
_acch_spec = pl.BlockSpec((NC, _RB, HH), lambda i: (0, i, 0))
_row_spec = pl.BlockSpec((_RB, H), lambda i: (i, 0))

_y = pl.pallas_call(
    _y_body,
    grid=(N // _RB,),
    in_specs=[_full((1, 1)), _acch_spec, _degp_spec, _row_spec, _full((1, H)),
              _full((H, H))],
    out_specs=_acch_spec,
    out_shape=jax.ShapeDtypeStruct((NC, N, HH), jnp.float32),
)

_z = pl.pallas_call(
    _z_body,
    grid=(N // _RB,),
    in_specs=[_acch_spec, _degp_spec, _full((1, H)), _full((H, 64)),
              _full((1, 64))],
    out_specs=pl.BlockSpec((_RB, 64), lambda i: (i, 0)),
    out_shape=jax.ShapeDtypeStruct((N, 64), jnp.float32),
)


def kernel(x, edge_index, W1, b1, W2, b2, Wl, bl):
    src = edge_index[0]
    dst = edge_index[1]
    degp = _deg_partials(dst)

    ws = jnp.stack([W1, W2])
    bs = jnp.stack([jnp.zeros((1, H), jnp.float32), b1.reshape(1, H)])
    flags = jnp.arange(2, dtype=jnp.float32).reshape(2, 1, 1)

    def step(acch, xs):
        w, b, flag = xs
        yl = _y(flag, acch, degp, x, b, w)
        return _agg_halves(yl, src, dst), None

    acch0 = jnp.zeros((NC, N, HH), jnp.float32)
    acch, _ = lax.scan(step, acch0, (ws, bs, flags))
    return _z(acch, degp, b2.reshape(1, H), Wl, bl.reshape(1, 64))


# speedup vs baseline: 22.5447x; 2.5905x over previous
import functools

import jax
import jax.numpy as jnp
from jax import lax
from jax.experimental import pallas as pl
from jax.experimental.pallas import tpu as pltpu
from jax.experimental.pallas import tpu_sc as plsc

N = 10000
E = 320000
H = 128
NC = 2
NS = 16
NW = NC * NS
EPW = E // NW
CH = 80
NCHUNK = EPW // CH
RSTRIDE = 624
RLEN = 640

_mesh = plsc.VectorSubcoreMesh(core_axis_name="c", subcore_axis_name="s")


def _zero_fill(buf, nrows, ncols, value=0.0):
    vec = jnp.full((16,), value, dtype=jnp.float32)

    def body(i, _):
        for j in range(ncols // 16):
            buf[i, pl.ds(j * 16, 16)] = vec
        return 0

    lax.fori_loop(0, nrows, body, 0)


def _deg_body(dst_hbm, out_hbm, dacc, dst_buf, ones80, initbuf, gsem):
    c = lax.axis_index("c")
    s = lax.axis_index("s")
    wid = s * NC + c
    base = pl.multiple_of(s * RSTRIDE, 8)

    initval = jnp.where(c == 0, 1.0, 0.0)
    _zero_fill(initbuf, RLEN, 16, initval)
    _zero_fill(ones80, CH, 16, 1.0)
    pltpu.sync_copy(dst_hbm.at[pl.ds(wid * NCHUNK, NCHUNK)], dst_buf)
    pltpu.sync_copy(initbuf, dacc.at[pl.ds(base, RLEN)])
    plsc.subcore_barrier()

    def body(i, _):
        pltpu.sync_copy(ones80, dacc.at[dst_buf.at[i]], add=True)
        return 0

    lax.fori_loop(0, NCHUNK, body, 0)
    plsc.subcore_barrier()
    pltpu.sync_copy(dacc.at[pl.ds(base, RLEN)], out_hbm.at[c, pl.ds(base, RLEN)])


_deg_partials = pl.kernel(
    _deg_body,
    out_type=jax.ShapeDtypeStruct((NC, N, 16), jnp.float32),
    mesh=_mesh,
    scratch_types=[
        pltpu.VMEM_SHARED((N, 16), jnp.float32),
        pltpu.VMEM((NCHUNK, CH), jnp.int32),
        pltpu.VMEM((CH, 16), jnp.float32),
        pltpu.VMEM((RLEN, 16), jnp.float32),
        pltpu.SemaphoreType.DMA,
    ],
    compiler_params=pltpu.CompilerParams(use_tc_tiling_on_sc=False),
)


HH = H // 2
EPT = E // NS
NCHUNK2 = EPT // CH


NBUF = 2


def _agg_body(y_hbm, src_hbm, dst_hbm, out_hbm, acc, src_buf, dst_buf, rows,
              gsem0, gsem1):
    c = lax.axis_index("c")
    s = lax.axis_index("s")
    base = pl.multiple_of(s * RSTRIDE, 8)
    row0 = s * NCHUNK2

    pltpu.sync_copy(src_hbm.at[pl.ds(row0, NCHUNK2)], src_buf)
    pltpu.sync_copy(dst_hbm.at[pl.ds(row0, NCHUNK2)], dst_buf)
    pltpu.sync_copy(y_hbm.at[c, pl.ds(base, RLEN)], acc.at[pl.ds(base, RLEN)])
    plsc.subcore_barrier()

    yc = y_hbm.at[c]
    sems = (gsem0, gsem1)

    pltpu.async_copy(yc.at[src_buf.at[0]], rows.at[0], sems[0])

    def outer(k, _):
        for b in range(NBUF):
            chunk = NBUF * k + b
            nxt_b = (b + 1) % NBUF

            @pl.when(chunk + 1 < NCHUNK2)
            def _():
                pltpu.async_copy(yc.at[src_buf.at[chunk + 1]], rows.at[nxt_b],
                                 sems[nxt_b])

            pltpu.make_async_copy(yc.at[src_buf.at[chunk]], rows.at[b],
                                  sems[b]).wait()
            pltpu.sync_copy(rows.at[b], acc.at[dst_buf.at[chunk]], add=True)
        return 0

    lax.fori_loop(0, NCHUNK2 // NBUF, outer, 0)
    plsc.subcore_barrier()
    pltpu.sync_copy(acc.at[pl.ds(base, RLEN)], out_hbm.at[c, pl.ds(base, RLEN)])


_agg_halves = pl.kernel(
    _agg_body,
    out_type=jax.ShapeDtypeStruct((NC, N, HH), jnp.float32),
    mesh=_mesh,
    scratch_types=[
        pltpu.VMEM_SHARED((N, HH), jnp.float32),
        pltpu.VMEM((NCHUNK2, CH), jnp.int32),
        pltpu.VMEM((NCHUNK2, CH), jnp.int32),
        pltpu.VMEM((NBUF, CH, HH), jnp.float32),
        pltpu.SemaphoreType.DMA,
        pltpu.SemaphoreType.DMA,
    ],
    compiler_params=pltpu.CompilerParams(use_tc_tiling_on_sc=False),
)


_RB = 1000


def _dis_from(degp):
    deg = degp[0] + degp[1]
    return lax.rsqrt(jnp.maximum(deg[:, 0:1], 1.0))


def _y_body(flag_ref, acch_ref, degp_ref, x_ref, b_ref, w_ref, y_ref):
    dis = _dis_from(degp_ref)
    a = jnp.concatenate([acch_ref[0], acch_ref[1]], axis=1)
    h = jnp.maximum(a * dis + b_ref[...], 0.0)
    pre = jnp.where(flag_ref[...] > 0, h, x_ref[...])
    y = jnp.dot(pre, w_ref[...], preferred_element_type=jnp.float32) * dis
    y_ref[0] = y[:, :HH]
    y_ref[1] = y[:, HH:]


def _z_body(acch_ref, degp_ref, b2_ref, wl_ref, bl_ref, z_ref):
    dis = _dis_from(degp_ref)
    a = jnp.concatenate([acch_ref[0], acch_ref[1]], axis=1)
    h = jnp.maximum(a * dis + b2_ref[...], 0.0)
    z_ref[...] = jnp.dot(h, wl_ref[...],
                         preferred_element_type=jnp.float32) + bl_ref[...]


def _full(shape):
    return pl.BlockSpec(shape, lambda i: (0,) * len(shape))


_degp_spec = pl.BlockSpec((NC, _RB, 16), lambda i: (0, i, 0))
_acch_spec = pl.BlockSpec((NC, _RB, HH), lambda i: (0, i, 0))
_row_spec = pl.BlockSpec((_RB, H), lambda i: (i, 0))

_y = pl.pallas_call(
    _y_body,
    grid=(N // _RB,),
    in_specs=[_full((1, 1)), _acch_spec, _degp_spec, _row_spec, _full((1, H)),
              _full((H, H))],
    out_specs=_acch_spec,
    out_shape=jax.ShapeDtypeStruct((NC, N, HH), jnp.float32),
)

_z = pl.pallas_call(
    _z_body,
    grid=(N // _RB,),
    in_specs=[_acch_spec, _degp_spec, _full((1, H)), _full((H, 64)),
              _full((1, 64))],
    out_specs=pl.BlockSpec((_RB, 64), lambda i: (i, 0)),
    out_shape=jax.ShapeDtypeStruct((N, 64), jnp.float32),
)


def kernel(x, edge_index, W1, b1, W2, b2, Wl, bl):
    src = edge_index[0].reshape(E // CH, CH)
    dst = edge_index[1].reshape(E // CH, CH)
    degp = _deg_partials(dst)

    ws = jnp.stack([W1, W2])
    bs = jnp.stack([jnp.zeros((1, H), jnp.float32), b1.reshape(1, H)])
    flags = jnp.arange(2, dtype=jnp.float32).reshape(2, 1, 1)

    def step(acch, xs):
        w, b, flag = xs
        yl = _y(flag, acch, degp, x, b, w)
        return _agg_halves(yl, src, dst), None

    acch0 = jnp.zeros((NC, N, HH), jnp.float32)
    acch, _ = lax.scan(step, acch0, (ws, bs, flags))
    return _z(acch, degp, b2.reshape(1, H), Wl, bl.reshape(1, 64))


# speedup vs baseline: 29.8414x; 1.3237x over previous
import jax
import jax.numpy as jnp
from jax import lax
from jax.experimental import pallas as pl
from jax.experimental.pallas import tpu as pltpu
from jax.experimental.pallas import tpu_sc as plsc

N = 10000
E = 320000
H = 128
HH = H // 2
NC = 2
NS = 16
NW = NC * NS
CH = 100
EPW = E // NW
NCHUNK = EPW // CH
EPT = E // NS
NCHUNK2 = EPT // CH

RSTRIDE = 624
RLEN = 640

_mesh = plsc.VectorSubcoreMesh(core_axis_name="c", subcore_axis_name="s")


def _fill(buf, nrows, ncols, value):
    vec = jnp.full((16,), value, dtype=jnp.float32)

    def body(i, _):
        for j in range(ncols // 16):
            buf[i, pl.ds(j * 16, 16)] = vec
        return 0

    lax.fori_loop(0, nrows, body, 0)


def _deg_body(dst_hbm, out_hbm, dacc, dst_buf, ones, initbuf, gsem):
    c = lax.axis_index("c")
    s = lax.axis_index("s")
    wid = s * NC + c
    base = pl.multiple_of(s * RSTRIDE, 8)

    initval = jnp.where(c == 0, 1.0, 0.0)
    _fill(initbuf, RLEN, 16, initval)
    _fill(ones, CH, 16, 1.0)
    pltpu.sync_copy(dst_hbm.at[pl.ds(wid * NCHUNK, NCHUNK)], dst_buf)
    pltpu.sync_copy(initbuf, dacc.at[pl.ds(base, RLEN)])
    plsc.subcore_barrier()

    def body(i, _):
        pltpu.sync_copy(ones, dacc.at[dst_buf.at[i]], add=True)
        return 0

    lax.fori_loop(0, NCHUNK, body, 0)
    plsc.subcore_barrier()
    pltpu.sync_copy(dacc.at[pl.ds(base, RLEN)], out_hbm.at[c, pl.ds(base, RLEN)])


_deg_partials = pl.kernel(
    _deg_body,
    out_type=jax.ShapeDtypeStruct((NC, N, 16), jnp.float32),
    mesh=_mesh,
    scratch_types=[
        pltpu.VMEM_SHARED((N, 16), jnp.float32),
        pltpu.VMEM((NCHUNK, CH), jnp.int32),
        pltpu.VMEM((CH, 16), jnp.float32),
        pltpu.VMEM((RLEN, 16), jnp.float32),
        pltpu.SemaphoreType.DMA,
    ],
    compiler_params=pltpu.CompilerParams(use_tc_tiling_on_sc=False),
)


NBUF = 4
ADV = 2


def _agg_body(y_hbm, src_hbm, dst_hbm, out_hbm, acc, src_buf, dst_buf, rows,
              *sems):
    c = lax.axis_index("c")
    s = lax.axis_index("s")
    base = pl.multiple_of(s * RSTRIDE, 8)
    row0 = s * NCHUNK2
    gsem = sems[:NBUF]
    ssem = sems[NBUF:]

    pltpu.sync_copy(src_hbm.at[pl.ds(row0, NCHUNK2)], src_buf)
    pltpu.sync_copy(dst_hbm.at[pl.ds(row0, NCHUNK2)], dst_buf)
    pltpu.sync_copy(y_hbm.at[c, pl.ds(base, RLEN)], acc.at[pl.ds(base, RLEN)])
    plsc.subcore_barrier()

    yc = y_hbm.at[c]

    for p in range(ADV):
        pltpu.async_copy(yc.at[src_buf.at[p]], rows.at[p], gsem[p])

    def outer(k, _):
        for b in range(NBUF):
            chunk = NBUF * k + b
            nb = (b + ADV) % NBUF

            @pl.when(chunk + ADV < NCHUNK2)
            def _():
                @pl.when(chunk + ADV >= NBUF)
                def _():
                    pltpu.make_async_copy(rows.at[nb], acc.at[dst_buf.at[0]],
                                          ssem[nb]).wait()
                pltpu.async_copy(yc.at[src_buf.at[chunk + ADV]], rows.at[nb],
                                 gsem[nb])

            pltpu.make_async_copy(yc.at[src_buf.at[chunk]], rows.at[b],
                                  gsem[b]).wait()
            pltpu.async_copy(rows.at[b], acc.at[dst_buf.at[chunk]], ssem[b],
                             add=True)
        return 0

    lax.fori_loop(0, NCHUNK2 // NBUF, outer, 0)
    for b in range(NBUF):
        pltpu.make_async_copy(rows.at[b], acc.at[dst_buf.at[0]],
                              ssem[b]).wait()
    plsc.subcore_barrier()
    pltpu.sync_copy(acc.at[pl.ds(base, RLEN)], out_hbm.at[c, pl.ds(base, RLEN)])


_agg_halves = pl.kernel(
    _agg_body,
    out_type=jax.ShapeDtypeStruct((NC, N, HH), jnp.float32),
    mesh=_mesh,
    scratch_types=[
        pltpu.VMEM_SHARED((N, HH), jnp.float32),
        pltpu.VMEM((NCHUNK2, CH), jnp.int32),
        pltpu.VMEM((NCHUNK2, CH), jnp.int32),
        pltpu.VMEM((NBUF, CH, HH), jnp.float32),
    ] + [pltpu.SemaphoreType.DMA] * (2 * NBUF),
    compiler_params=pltpu.CompilerParams(use_tc_tiling_on_sc=False),
)


_RB = 1000


def _dis_from(degp):
    deg = degp[0] + degp[1]
    return lax.rsqrt(jnp.maximum(deg[:, 0:1], 1.0))


def _y1_body(degp_ref, x_ref, w_ref, y_ref):
    dis = _dis_from(degp_ref)
    y = jnp.dot(x_ref[...], w_ref[...], preferred_element_type=jnp.float32) * dis
    y_ref[0] = y[:, :HH]
    y_ref[1] = y[:, HH:]


def _y2_body(acch_ref, degp_ref, b_ref, w_ref, y_ref):
    dis = _dis_from(degp_ref)
    a = jnp.concatenate([acch_ref[0], acch_ref[1]], axis=1)
    h = jnp.maximum(a * dis + b_ref[...], 0.0)
    y = jnp.dot(h, w_ref[...], preferred_element_type=jnp.float32) * dis
    y_ref[0] = y[:, :HH]
    y_ref[1] = y[:, HH:]


def _z_body(acch_ref, degp_ref, b2_ref, wl_ref, bl_ref, z_ref):
    dis = _dis_from(degp_ref)
    a = jnp.concatenate([acch_ref[0], acch_ref[1]], axis=1)
    h = jnp.maximum(a * dis + b2_ref[...], 0.0)
    z_ref[...] = jnp.dot(h, wl_ref[...],
                         preferred_element_type=jnp.float32) + bl_ref[...]


def _full(shape):
    return pl.BlockSpec(shape, lambda i: (0,) * len(shape))


_degp_spec = pl.BlockSpec((NC, _RB, 16), lambda i: (0, i, 0))
_acch_spec = pl.BlockSpec((NC, _RB, HH), lambda i: (0, i, 0))
_row_spec = pl.BlockSpec((_RB, H), lambda i: (i, 0))

_y1 = pl.pallas_call(
    _y1_body,
    grid=(N // _RB,),
    in_specs=[_degp_spec, _row_spec, _full((H, H))],
    out_specs=_acch_spec,
    out_shape=jax.ShapeDtypeStruct((NC, N, HH), jnp.float32),
)

_y2 = pl.pallas_call(
    _y2_body,
    grid=(N // _RB,),
    in_specs=[_acch_spec, _degp_spec, _full((1, H)), _full((H, H))],
    out_specs=_acch_spec,
    out_shape=jax.ShapeDtypeStruct((NC, N, HH), jnp.float32),
)

_z = pl.pallas_call(
    _z_body,
    grid=(N // _RB,),
    in_specs=[_acch_spec, _degp_spec, _full((1, H)), _full((H, 64)),
              _full((1, 64))],
    out_specs=pl.BlockSpec((_RB, 64), lambda i: (i, 0)),
    out_shape=jax.ShapeDtypeStruct((N, 64), jnp.float32),
)


def kernel(x, edge_index, W1, b1, W2, b2, Wl, bl):
    src = edge_index[0].reshape(E // CH, CH)
    dst = edge_index[1].reshape(E // CH, CH)
    degp = _deg_partials(dst)
    y1 = _y1(degp, x, W1)
    a1 = _agg_halves(y1, src, dst)
    y2 = _y2(a1, degp, b1.reshape(1, H), W2)
    a2 = _agg_halves(y2, src, dst)
    return _z(a2, degp, b2.reshape(1, H), Wl, bl.reshape(1, 64))


# speedup vs baseline: 30.0740x; 1.0078x over previous
import jax
import jax.numpy as jnp
from jax import lax
from jax.experimental import pallas as pl
from jax.experimental.pallas import tpu as pltpu
from jax.experimental.pallas import tpu_sc as plsc

N = 10000
E = 320000
H = 128
HH = H // 2
NC = 2
NS = 16
NW = NC * NS
CH = 125
EPW = E // NW
NCHUNK = EPW // CH
EPT = E // NS
NCHUNK2 = EPT // CH

RSTRIDE = 624
RLEN = 640

_mesh = plsc.VectorSubcoreMesh(core_axis_name="c", subcore_axis_name="s")


def _fill(buf, nrows, ncols, value):
    vec = jnp.full((16,), value, dtype=jnp.float32)

    def body(i, _):
        for j in range(ncols // 16):
            buf[i, pl.ds(j * 16, 16)] = vec
        return 0

    lax.fori_loop(0, nrows, body, 0)


def _deg_body(dst_hbm, out_hbm, dacc, dst_buf, ones, initbuf, gsem):
    c = lax.axis_index("c")
    s = lax.axis_index("s")
    wid = s * NC + c
    base = pl.multiple_of(s * RSTRIDE, 8)

    initval = jnp.where(c == 0, 1.0, 0.0)
    _fill(initbuf, RLEN, 16, initval)
    _fill(ones, CH, 16, 1.0)
    pltpu.sync_copy(dst_hbm.at[pl.ds(wid * NCHUNK, NCHUNK)], dst_buf)
    pltpu.sync_copy(initbuf, dacc.at[pl.ds(base, RLEN)])
    plsc.subcore_barrier()

    def body(i, _):
        pltpu.sync_copy(ones, dacc.at[dst_buf.at[i]], add=True)
        return 0

    lax.fori_loop(0, NCHUNK, body, 0)
    plsc.subcore_barrier()
    pltpu.sync_copy(dacc.at[pl.ds(base, RLEN)], out_hbm.at[c, pl.ds(base, RLEN)])


_deg_partials = pl.kernel(
    _deg_body,
    out_type=jax.ShapeDtypeStruct((NC, N, 16), jnp.float32),
    mesh=_mesh,
    scratch_types=[
        pltpu.VMEM_SHARED((N, 16), jnp.float32),
        pltpu.VMEM((NCHUNK, CH), jnp.int32),
        pltpu.VMEM((CH, 16), jnp.float32),
        pltpu.VMEM((RLEN, 16), jnp.float32),
        pltpu.SemaphoreType.DMA,
    ],
    compiler_params=pltpu.CompilerParams(use_tc_tiling_on_sc=False),
)


NBUF = 4
ADV = 2


def _agg_body(y_hbm, src_hbm, dst_hbm, out_hbm, acc, src_buf, dst_buf, rows,
              *sems):
    c = lax.axis_index("c")
    s = lax.axis_index("s")
    base = pl.multiple_of(s * RSTRIDE, 8)
    row0 = s * NCHUNK2
    gsem = sems[:NBUF]
    ssem = sems[NBUF:]

    pltpu.sync_copy(src_hbm.at[pl.ds(row0, NCHUNK2)], src_buf)
    pltpu.sync_copy(dst_hbm.at[pl.ds(row0, NCHUNK2)], dst_buf)
    pltpu.sync_copy(y_hbm.at[c, pl.ds(base, RLEN)], acc.at[pl.ds(base, RLEN)])
    plsc.subcore_barrier()

    yc = y_hbm.at[c]

    for p in range(ADV):
        pltpu.async_copy(yc.at[src_buf.at[p]], rows.at[p], gsem[p])

    def outer(k, _):
        for b in range(NBUF):
            chunk = NBUF * k + b
            nb = (b + ADV) % NBUF

            @pl.when(chunk + ADV < NCHUNK2)
            def _():
                @pl.when(chunk + ADV >= NBUF)
                def _():
                    pltpu.make_async_copy(rows.at[nb], acc.at[dst_buf.at[0]],
                                          ssem[nb]).wait()
                pltpu.async_copy(yc.at[src_buf.at[chunk + ADV]], rows.at[nb],
                                 gsem[nb])

            pltpu.make_async_copy(yc.at[src_buf.at[chunk]], rows.at[b],
                                  gsem[b]).wait()
            pltpu.async_copy(rows.at[b], acc.at[dst_buf.at[chunk]], ssem[b],
                             add=True)
        return 0

    lax.fori_loop(0, NCHUNK2 // NBUF, outer, 0)
    for b in range(NBUF):
        pltpu.make_async_copy(rows.at[b], acc.at[dst_buf.at[0]],
                              ssem[b]).wait()
    plsc.subcore_barrier()
    pltpu.sync_copy(acc.at[pl.ds(base, RLEN)], out_hbm.at[c, pl.ds(base, RLEN)])


_agg_halves = pl.kernel(
    _agg_body,
    out_type=jax.ShapeDtypeStruct((NC, N, HH), jnp.float32),
    mesh=_mesh,
    scratch_types=[
        pltpu.VMEM_SHARED((N, HH), jnp.float32),
        pltpu.VMEM((NCHUNK2, CH), jnp.int32),
        pltpu.VMEM((NCHUNK2, CH), jnp.int32),
        pltpu.VMEM((NBUF, CH, HH), jnp.float32),
    ] + [pltpu.SemaphoreType.DMA] * (2 * NBUF),
    compiler_params=pltpu.CompilerParams(use_tc_tiling_on_sc=False),
)


_RB = 1000


def _dis_from(degp):
    deg = degp[0] + degp[1]
    return lax.rsqrt(jnp.maximum(deg[:, 0:1], 1.0))


def _y1_body(degp_ref, x_ref, w_ref, y_ref):
    dis = _dis_from(degp_ref)
    y = jnp.dot(x_ref[...], w_ref[...], preferred_element_type=jnp.float32) * dis
    y_ref[0] = y[:, :HH]
    y_ref[1] = y[:, HH:]


def _y2_body(acch_ref, degp_ref, b_ref, w_ref, y_ref):
    dis = _dis_from(degp_ref)
    a = jnp.concatenate([acch_ref[0], acch_ref[1]], axis=1)
    h = jnp.maximum(a * dis + b_ref[...], 0.0)
    y = jnp.dot(h, w_ref[...], preferred_element_type=jnp.float32) * dis
    y_ref[0] = y[:, :HH]
    y_ref[1] = y[:, HH:]


def _z_body(acch_ref, degp_ref, b2_ref, wl_ref, bl_ref, z_ref):
    dis = _dis_from(degp_ref)
    a = jnp.concatenate([acch_ref[0], acch_ref[1]], axis=1)
    h = jnp.maximum(a * dis + b2_ref[...], 0.0)
    z_ref[...] = jnp.dot(h, wl_ref[...],
                         preferred_element_type=jnp.float32) + bl_ref[...]


def _full(shape):
    return pl.BlockSpec(shape, lambda i: (0,) * len(shape))


_degp_spec = pl.BlockSpec((NC, _RB, 16), lambda i: (0, i, 0))
_acch_spec = pl.BlockSpec((NC, _RB, HH), lambda i: (0, i, 0))
_row_spec = pl.BlockSpec((_RB, H), lambda i: (i, 0))

_y1 = pl.pallas_call(
    _y1_body,
    grid=(N // _RB,),
    in_specs=[_degp_spec, _row_spec, _full((H, H))],
    out_specs=_acch_spec,
    out_shape=jax.ShapeDtypeStruct((NC, N, HH), jnp.float32),
)

_y2 = pl.pallas_call(
    _y2_body,
    grid=(N // _RB,),
    in_specs=[_acch_spec, _degp_spec, _full((1, H)), _full((H, H))],
    out_specs=_acch_spec,
    out_shape=jax.ShapeDtypeStruct((NC, N, HH), jnp.float32),
)

_z = pl.pallas_call(
    _z_body,
    grid=(N // _RB,),
    in_specs=[_acch_spec, _degp_spec, _full((1, H)), _full((H, 64)),
              _full((1, 64))],
    out_specs=pl.BlockSpec((_RB, 64), lambda i: (i, 0)),
    out_shape=jax.ShapeDtypeStruct((N, 64), jnp.float32),
)


def kernel(x, edge_index, W1, b1, W2, b2, Wl, bl):
    src = edge_index[0].reshape(E // CH, CH)
    dst = edge_index[1].reshape(E // CH, CH)
    degp = _deg_partials(dst)
    y1 = _y1(degp, x, W1)
    a1 = _agg_halves(y1, src, dst)
    y2 = _y2(a1, degp, b1.reshape(1, H), W2)
    a2 = _agg_halves(y2, src, dst)
    return _z(a2, degp, b2.reshape(1, H), Wl, bl.reshape(1, 64))


# speedup vs baseline: 31.4127x; 1.0445x over previous
import jax
import jax.numpy as jnp
from jax import lax
from jax.experimental import pallas as pl
from jax.experimental.pallas import tpu as pltpu
from jax.experimental.pallas import tpu_sc as plsc

N = 10000
E = 320000
H = 128
HH = H // 2
NC = 2
NS = 16
NW = NC * NS
CH = 125
EPW = E // NW
NCHUNK = EPW // CH
EPT = E // NS
NCHUNK2 = EPT // CH

RSTRIDE = 624
RLEN = 640

_mesh = plsc.VectorSubcoreMesh(core_axis_name="c", subcore_axis_name="s")


def _fill(buf, nrows, ncols, value):
    vec = jnp.full((16,), value, dtype=jnp.float32)

    def body(i, _):
        for j in range(ncols // 16):
            buf[i, pl.ds(j * 16, 16)] = vec
        return 0

    lax.fori_loop(0, nrows, body, 0)


def _deg_body(dst_hbm, out_hbm, dacc, dst_buf, ones, initbuf, gsem):
    c = lax.axis_index("c")
    s = lax.axis_index("s")
    wid = s * NC + c
    base = pl.multiple_of(s * RSTRIDE, 8)

    initval = jnp.where(c == 0, 1.0, 0.0)
    _fill(initbuf, RLEN, 16, initval)
    _fill(ones, CH, 16, 1.0)
    pltpu.sync_copy(dst_hbm.at[pl.ds(wid * NCHUNK, NCHUNK)], dst_buf)
    pltpu.sync_copy(initbuf, dacc.at[pl.ds(base, RLEN)])
    plsc.subcore_barrier()

    def body(i, _):
        pltpu.sync_copy(ones, dacc.at[dst_buf.at[i]], add=True)
        return 0

    lax.fori_loop(0, NCHUNK, body, 0)
    plsc.subcore_barrier()
    pltpu.sync_copy(dacc.at[pl.ds(base, RLEN)], out_hbm.at[c, pl.ds(base, RLEN)])


_deg_partials = pl.kernel(
    _deg_body,
    out_type=jax.ShapeDtypeStruct((NC, N, 16), jnp.float32),
    mesh=_mesh,
    scratch_types=[
        pltpu.VMEM_SHARED((N, 16), jnp.float32),
        pltpu.VMEM((NCHUNK, CH), jnp.int32),
        pltpu.VMEM((CH, 16), jnp.float32),
        pltpu.VMEM((RLEN, 16), jnp.float32),
        pltpu.SemaphoreType.DMA,
    ],
    compiler_params=pltpu.CompilerParams(use_tc_tiling_on_sc=False),
)


NBUF = 5
ADV = 2


def _agg_body(y_hbm, src_hbm, dst_hbm, out_hbm, acc, src_buf, dst_buf, rows,
              *sems):
    c = lax.axis_index("c")
    s = lax.axis_index("s")
    base = pl.multiple_of(s * RSTRIDE, 8)
    row0 = s * NCHUNK2
    gsem = sems[:NBUF]
    ssem = sems[NBUF:]

    pltpu.sync_copy(src_hbm.at[pl.ds(row0, NCHUNK2)], src_buf)
    pltpu.sync_copy(dst_hbm.at[pl.ds(row0, NCHUNK2)], dst_buf)
    pltpu.sync_copy(y_hbm.at[c, pl.ds(base, RLEN)], acc.at[pl.ds(base, RLEN)])
    plsc.subcore_barrier()

    yc = y_hbm.at[c]

    for p in range(ADV):
        pltpu.async_copy(yc.at[src_buf.at[p]], rows.at[p], gsem[p])

    def outer(k, _):
        for b in range(NBUF):
            chunk = NBUF * k + b
            nb = (b + ADV) % NBUF

            @pl.when(chunk + ADV < NCHUNK2)
            def _():
                @pl.when(chunk + ADV >= NBUF)
                def _():
                    pltpu.make_async_copy(rows.at[nb], acc.at[dst_buf.at[0]],
                                          ssem[nb]).wait()
                pltpu.async_copy(yc.at[src_buf.at[chunk + ADV]], rows.at[nb],
                                 gsem[nb])

            pltpu.make_async_copy(yc.at[src_buf.at[chunk]], rows.at[b],
                                  gsem[b]).wait()
            pltpu.async_copy(rows.at[b], acc.at[dst_buf.at[chunk]], ssem[b],
                             add=True)
        return 0

    lax.fori_loop(0, NCHUNK2 // NBUF, outer, 0)
    for b in range(NBUF):
        pltpu.make_async_copy(rows.at[b], acc.at[dst_buf.at[0]],
                              ssem[b]).wait()
    plsc.subcore_barrier()
    pltpu.sync_copy(acc.at[pl.ds(base, RLEN)], out_hbm.at[c, pl.ds(base, RLEN)])


_agg_halves = pl.kernel(
    _agg_body,
    out_type=jax.ShapeDtypeStruct((NC, N, HH), jnp.float32),
    mesh=_mesh,
    scratch_types=[
        pltpu.VMEM_SHARED((N, HH), jnp.float32),
        pltpu.VMEM((NCHUNK2, CH), jnp.int32),
        pltpu.VMEM((NCHUNK2, CH), jnp.int32),
        pltpu.VMEM((NBUF, CH, HH), jnp.float32),
    ] + [pltpu.SemaphoreType.DMA] * (2 * NBUF),
    compiler_params=pltpu.CompilerParams(use_tc_tiling_on_sc=False),
)


_RB = 2000


def _dis_from(degp):
    deg = degp[0] + degp[1]
    return lax.rsqrt(jnp.maximum(deg[:, 0:1], 1.0))


def _y1_body(degp_ref, x_ref, w_ref, y_ref):
    dis = _dis_from(degp_ref)
    y = jnp.dot(x_ref[...], w_ref[...], preferred_element_type=jnp.float32) * dis
    y_ref[0] = y[:, :HH]
    y_ref[1] = y[:, HH:]


def _y2_body(acch_ref, degp_ref, b_ref, w_ref, y_ref):
    dis = _dis_from(degp_ref)
    a = jnp.concatenate([acch_ref[0], acch_ref[1]], axis=1)
    h = jnp.maximum(a * dis + b_ref[...], 0.0)
    y = jnp.dot(h, w_ref[...], preferred_element_type=jnp.float32) * dis
    y_ref[0] = y[:, :HH]
    y_ref[1] = y[:, HH:]


def _z_body(acch_ref, degp_ref, b2_ref, wl_ref, bl_ref, z_ref):
    dis = _dis_from(degp_ref)
    a = jnp.concatenate([acch_ref[0], acch_ref[1]], axis=1)
    h = jnp.maximum(a * dis + b2_ref[...], 0.0)
    z_ref[...] = jnp.dot(h, wl_ref[...],
                         preferred_element_type=jnp.float32) + bl_ref[...]


def _full(shape):
    return pl.BlockSpec(shape, lambda i: (0,) * len(shape))


_degp_spec = pl.BlockSpec((NC, _RB, 16), lambda i: (0, i, 0))
_acch_spec = pl.BlockSpec((NC, _RB, HH), lambda i: (0, i, 0))
_row_spec = pl.BlockSpec((_RB, H), lambda i: (i, 0))

_y1 = pl.pallas_call(
    _y1_body,
    grid=(N // _RB,),
    in_specs=[_degp_spec, _row_spec, _full((H, H))],
    out_specs=_acch_spec,
    out_shape=jax.ShapeDtypeStruct((NC, N, HH), jnp.float32),
)

_y2 = pl.pallas_call(
    _y2_body,
    grid=(N // _RB,),
    in_specs=[_acch_spec, _degp_spec, _full((1, H)), _full((H, H))],
    out_specs=_acch_spec,
    out_shape=jax.ShapeDtypeStruct((NC, N, HH), jnp.float32),
)

_z = pl.pallas_call(
    _z_body,
    grid=(N // _RB,),
    in_specs=[_acch_spec, _degp_spec, _full((1, H)), _full((H, 64)),
              _full((1, 64))],
    out_specs=pl.BlockSpec((_RB, 64), lambda i: (i, 0)),
    out_shape=jax.ShapeDtypeStruct((N, 64), jnp.float32),
)


def kernel(x, edge_index, W1, b1, W2, b2, Wl, bl):
    src = edge_index[0].reshape(E // CH, CH)
    dst = edge_index[1].reshape(E // CH, CH)
    degp = _deg_partials(dst)
    y1 = _y1(degp, x, W1)
    a1 = _agg_halves(y1, src, dst)
    y2 = _y2(a1, degp, b1.reshape(1, H), W2)
    a2 = _agg_halves(y2, src, dst)
    return _z(a2, degp, b2.reshape(1, H), Wl, bl.reshape(1, 64))


# speedup vs baseline: 32.2781x; 1.0276x over previous
import jax
import jax.numpy as jnp
from jax import lax
from jax.experimental import pallas as pl
from jax.experimental.pallas import tpu as pltpu
from jax.experimental.pallas import tpu_sc as plsc

N = 10000
E = 320000
H = 128
HH = H // 2
NC = 2
NS = 16
NW = NC * NS
CH = 125
EPW = E // NW
NCHUNK = EPW // CH
EPT = E // NS
NCHUNK2 = EPT // CH

RSTRIDE = 624
RLEN = 640

_mesh = plsc.VectorSubcoreMesh(core_axis_name="c", subcore_axis_name="s")


def _fill(buf, nrows, ncols, value):
    vec = jnp.full((16,), value, dtype=jnp.float32)

    def body(i, _):
        for j in range(ncols // 16):
            buf[i, pl.ds(j * 16, 16)] = vec
        return 0

    lax.fori_loop(0, nrows, body, 0)


def _deg_body(dst_hbm, out_hbm, dacc, dst_buf, ones, initbuf, gsem):
    c = lax.axis_index("c")
    s = lax.axis_index("s")
    wid = s * NC + c
    base = pl.multiple_of(s * RSTRIDE, 8)

    initval = jnp.where(c == 0, 1.0, 0.0)
    _fill(initbuf, RLEN, 16, initval)
    _fill(ones, CH, 16, 1.0)
    pltpu.sync_copy(dst_hbm.at[pl.ds(wid * NCHUNK, NCHUNK)], dst_buf)
    pltpu.sync_copy(initbuf, dacc.at[pl.ds(base, RLEN)])
    plsc.subcore_barrier()

    def body(k, _):
        for j in range(8):
            pltpu.async_copy(ones, dacc.at[dst_buf.at[8 * k + j]], gsem,
                             add=True)
        for j in range(8):
            pltpu.make_async_copy(ones, dacc.at[dst_buf.at[0]], gsem).wait()
        return 0

    lax.fori_loop(0, NCHUNK // 8, body, 0)
    plsc.subcore_barrier()
    pltpu.sync_copy(dacc.at[pl.ds(base, RLEN)], out_hbm.at[c, pl.ds(base, RLEN)])


_deg_partials = pl.kernel(
    _deg_body,
    out_type=jax.ShapeDtypeStruct((NC, N, 16), jnp.float32),
    mesh=_mesh,
    scratch_types=[
        pltpu.VMEM_SHARED((N, 16), jnp.float32),
        pltpu.VMEM((NCHUNK, CH), jnp.int32),
        pltpu.VMEM((CH, 16), jnp.float32),
        pltpu.VMEM((RLEN, 16), jnp.float32),
        pltpu.SemaphoreType.DMA,
    ],
    compiler_params=pltpu.CompilerParams(use_tc_tiling_on_sc=False),
)


NBUF = 5
ADV = 2


def _agg_body(y_hbm, src_hbm, dst_hbm, out_hbm, acc, src_buf, dst_buf, rows,
              *sems):
    c = lax.axis_index("c")
    s = lax.axis_index("s")
    base = pl.multiple_of(s * RSTRIDE, 8)
    row0 = s * NCHUNK2
    gsem = sems[:NBUF]
    ssem = sems[NBUF:]

    pltpu.sync_copy(src_hbm.at[pl.ds(row0, NCHUNK2)], src_buf)
    pltpu.sync_copy(dst_hbm.at[pl.ds(row0, NCHUNK2)], dst_buf)
    pltpu.sync_copy(y_hbm.at[c, pl.ds(base, RLEN)], acc.at[pl.ds(base, RLEN)])
    plsc.subcore_barrier()

    yc = y_hbm.at[c]

    for p in range(ADV):
        pltpu.async_copy(yc.at[src_buf.at[p]], rows.at[p], gsem[p])

    def outer(k, _):
        for b in range(NBUF):
            chunk = NBUF * k + b
            nb = (b + ADV) % NBUF

            @pl.when(chunk + ADV < NCHUNK2)
            def _():
                @pl.when(chunk + ADV >= NBUF)
                def _():
                    pltpu.make_async_copy(rows.at[nb], acc.at[dst_buf.at[0]],
                                          ssem[nb]).wait()
                pltpu.async_copy(yc.at[src_buf.at[chunk + ADV]], rows.at[nb],
                                 gsem[nb])

            pltpu.make_async_copy(yc.at[src_buf.at[chunk]], rows.at[b],
                                  gsem[b]).wait()
            pltpu.async_copy(rows.at[b], acc.at[dst_buf.at[chunk]], ssem[b],
                             add=True)
        return 0

    lax.fori_loop(0, NCHUNK2 // NBUF, outer, 0)
    for b in range(NBUF):
        pltpu.make_async_copy(rows.at[b], acc.at[dst_buf.at[0]],
                              ssem[b]).wait()
    plsc.subcore_barrier()
    pltpu.sync_copy(acc.at[pl.ds(base, RLEN)], out_hbm.at[c, pl.ds(base, RLEN)])


_agg_halves = pl.kernel(
    _agg_body,
    out_type=jax.ShapeDtypeStruct((NC, N, HH), jnp.float32),
    mesh=_mesh,
    scratch_types=[
        pltpu.VMEM_SHARED((N, HH), jnp.float32),
        pltpu.VMEM((NCHUNK2, CH), jnp.int32),
        pltpu.VMEM((NCHUNK2, CH), jnp.int32),
        pltpu.VMEM((NBUF, CH, HH), jnp.float32),
    ] + [pltpu.SemaphoreType.DMA] * (2 * NBUF),
    compiler_params=pltpu.CompilerParams(use_tc_tiling_on_sc=False),
)


_RB = 5000


def _dis_from(degp):
    deg = degp[0] + degp[1]
    return lax.rsqrt(jnp.maximum(deg[:, 0:1], 1.0))


def _y1_body(degp_ref, x_ref, w_ref, y_ref):
    dis = _dis_from(degp_ref)
    y = jnp.dot(x_ref[...], w_ref[...], preferred_element_type=jnp.float32) * dis
    y_ref[0] = y[:, :HH]
    y_ref[1] = y[:, HH:]


def _y2_body(acch_ref, degp_ref, b_ref, w_ref, y_ref):
    dis = _dis_from(degp_ref)
    a = jnp.concatenate([acch_ref[0], acch_ref[1]], axis=1)
    h = jnp.maximum(a * dis + b_ref[...], 0.0)
    y = jnp.dot(h, w_ref[...], preferred_element_type=jnp.float32) * dis
    y_ref[0] = y[:, :HH]
    y_ref[1] = y[:, HH:]


def _z_body(acch_ref, degp_ref, b2_ref, wl_ref, bl_ref, z_ref):
    dis = _dis_from(degp_ref)
    a = jnp.concatenate([acch_ref[0], acch_ref[1]], axis=1)
    h = jnp.maximum(a * dis + b2_ref[...], 0.0)
    z_ref[...] = jnp.dot(h, wl_ref[...],
                         preferred_element_type=jnp.float32) + bl_ref[...]


def _full(shape):
    return pl.BlockSpec(shape, lambda i: (0,) * len(shape))


_degp_spec = pl.BlockSpec((NC, _RB, 16), lambda i: (0, i, 0))
_acch_spec = pl.BlockSpec((NC, _RB, HH), lambda i: (0, i, 0))
_row_spec = pl.BlockSpec((_RB, H), lambda i: (i, 0))

_y1 = pl.pallas_call(
    _y1_body,
    grid=(N // _RB,),
    in_specs=[_degp_spec, _row_spec, _full((H, H))],
    out_specs=_acch_spec,
    out_shape=jax.ShapeDtypeStruct((NC, N, HH), jnp.float32),
)

_y2 = pl.pallas_call(
    _y2_body,
    grid=(N // _RB,),
    in_specs=[_acch_spec, _degp_spec, _full((1, H)), _full((H, H))],
    out_specs=_acch_spec,
    out_shape=jax.ShapeDtypeStruct((NC, N, HH), jnp.float32),
)

_z = pl.pallas_call(
    _z_body,
    grid=(N // _RB,),
    in_specs=[_acch_spec, _degp_spec, _full((1, H)), _full((H, 64)),
              _full((1, 64))],
    out_specs=pl.BlockSpec((_RB, 64), lambda i: (i, 0)),
    out_shape=jax.ShapeDtypeStruct((N, 64), jnp.float32),
)


def kernel(x, edge_index, W1, b1, W2, b2, Wl, bl):
    src = edge_index[0].reshape(E // CH, CH)
    dst = edge_index[1].reshape(E // CH, CH)
    degp = _deg_partials(dst)
    y1 = _y1(degp, x, W1)
    a1 = _agg_halves(y1, src, dst)
    y2 = _y2(a1, degp, b1.reshape(1, H), W2)
    a2 = _agg_halves(y2, src, dst)
    return _z(a2, degp, b2.reshape(1, H), Wl, bl.reshape(1, 64))


# speedup vs baseline: 33.0389x; 1.0236x over previous
import jax
import jax.numpy as jnp
from jax import lax
from jax.experimental import pallas as pl
from jax.experimental.pallas import tpu as pltpu
from jax.experimental.pallas import tpu_sc as plsc

N = 10000
E = 320000
H = 128
HH = H // 2
NC = 2
NS = 16
NW = NC * NS
CH = 125
EPW = E // NW
NCHUNK = EPW // CH
EPT = E // NS
NCHUNK2 = EPT // CH

RSTRIDE = 624
RLEN = 640

_mesh = plsc.VectorSubcoreMesh(core_axis_name="c", subcore_axis_name="s")


def _fill(buf, nrows, ncols, value):
    vec = jnp.full((16,), value, dtype=jnp.float32)

    def body(i, _):
        for j in range(ncols // 16):
            buf[i, pl.ds(j * 16, 16)] = vec
        return 0

    lax.fori_loop(0, nrows, body, 0)


def _deg_body(dst_hbm, out_hbm, dacc, dst_buf, ones, initbuf, gsem):
    c = lax.axis_index("c")
    s = lax.axis_index("s")
    wid = s * NC + c
    base = pl.multiple_of(s * RSTRIDE, 8)

    initval = jnp.where(c == 0, 1.0, 0.0)
    _fill(initbuf, RLEN, 16, initval)
    _fill(ones, CH, 16, 1.0)
    pltpu.sync_copy(dst_hbm.at[pl.ds(wid * NCHUNK, NCHUNK)], dst_buf)
    pltpu.sync_copy(initbuf, dacc.at[pl.ds(base, RLEN)])
    plsc.subcore_barrier()

    def body(k, _):
        for j in range(8):
            pltpu.async_copy(ones, dacc.at[dst_buf.at[8 * k + j]], gsem,
                             add=True)
        for j in range(8):
            pltpu.make_async_copy(ones, dacc.at[dst_buf.at[0]], gsem).wait()
        return 0

    lax.fori_loop(0, NCHUNK // 8, body, 0)
    plsc.subcore_barrier()
    pltpu.sync_copy(dacc.at[pl.ds(base, RLEN)], out_hbm.at[c, pl.ds(base, RLEN)])


_deg_partials = pl.kernel(
    _deg_body,
    out_type=jax.ShapeDtypeStruct((NC, N, 16), jnp.float32),
    mesh=_mesh,
    scratch_types=[
        pltpu.VMEM_SHARED((N, 16), jnp.float32),
        pltpu.VMEM((NCHUNK, CH), jnp.int32),
        pltpu.VMEM((CH, 16), jnp.float32),
        pltpu.VMEM((RLEN, 16), jnp.float32),
        pltpu.SemaphoreType.DMA,
    ],
    compiler_params=pltpu.CompilerParams(use_tc_tiling_on_sc=False),
)


NBUF = 5
ADV = 3


def _agg_body(y_hbm, src_hbm, dst_hbm, out_hbm, acc, src_buf, dst_buf, rows,
              *sems):
    c = lax.axis_index("c")
    s = lax.axis_index("s")
    base = pl.multiple_of(s * RSTRIDE, 8)
    row0 = s * NCHUNK2
    gsem = sems[:NBUF]
    ssem = sems[NBUF:]

    pltpu.sync_copy(src_hbm.at[pl.ds(row0, NCHUNK2)], src_buf)
    pltpu.sync_copy(dst_hbm.at[pl.ds(row0, NCHUNK2)], dst_buf)
    pltpu.sync_copy(y_hbm.at[c, pl.ds(base, RLEN)], acc.at[pl.ds(base, RLEN)])
    plsc.subcore_barrier()

    yc = y_hbm.at[c]

    for p in range(ADV):
        pltpu.async_copy(yc.at[src_buf.at[p]], rows.at[p], gsem[p])

    def outer(k, _):
        for b in range(NBUF):
            chunk = NBUF * k + b
            nb = (b + ADV) % NBUF

            @pl.when(chunk + ADV < NCHUNK2)
            def _():
                @pl.when(chunk + ADV >= NBUF)
                def _():
                    pltpu.make_async_copy(rows.at[nb], acc.at[dst_buf.at[0]],
                                          ssem[nb]).wait()
                pltpu.async_copy(yc.at[src_buf.at[chunk + ADV]], rows.at[nb],
                                 gsem[nb])

            pltpu.make_async_copy(yc.at[src_buf.at[chunk]], rows.at[b],
                                  gsem[b]).wait()
            pltpu.async_copy(rows.at[b], acc.at[dst_buf.at[chunk]], ssem[b],
                             add=True)
        return 0

    lax.fori_loop(0, NCHUNK2 // NBUF, outer, 0)
    for b in range(NBUF):
        pltpu.make_async_copy(rows.at[b], acc.at[dst_buf.at[0]],
                              ssem[b]).wait()
    plsc.subcore_barrier()
    pltpu.sync_copy(acc.at[pl.ds(base, RLEN)], out_hbm.at[c, pl.ds(base, RLEN)])


_agg_halves = pl.kernel(
    _agg_body,
    out_type=jax.ShapeDtypeStruct((NC, N, HH), jnp.float32),
    mesh=_mesh,
    scratch_types=[
        pltpu.VMEM_SHARED((N, HH), jnp.float32),
        pltpu.VMEM((NCHUNK2, CH), jnp.int32),
        pltpu.VMEM((NCHUNK2, CH), jnp.int32),
        pltpu.VMEM((NBUF, CH, HH), jnp.float32),
    ] + [pltpu.SemaphoreType.DMA] * (2 * NBUF),
    compiler_params=pltpu.CompilerParams(use_tc_tiling_on_sc=False),
)


_RB = 5000


def _dis_from(degp):
    deg = degp[0] + degp[1]
    return lax.rsqrt(jnp.maximum(deg[:, 0:1], 1.0))


def _y1_body(degp_ref, x_ref, w_ref, y_ref):
    dis = _dis_from(degp_ref)
    y = jnp.dot(x_ref[...], w_ref[...], preferred_element_type=jnp.float32) * dis
    y_ref[0] = y[:, :HH]
    y_ref[1] = y[:, HH:]


def _y2_body(acch_ref, degp_ref, b_ref, w_ref, y_ref):
    dis = _dis_from(degp_ref)
    a = jnp.concatenate([acch_ref[0], acch_ref[1]], axis=1)
    h = jnp.maximum(a * dis + b_ref[...], 0.0)
    y = jnp.dot(h, w_ref[...], preferred_element_type=jnp.float32) * dis
    y_ref[0] = y[:, :HH]
    y_ref[1] = y[:, HH:]


def _z_body(acch_ref, degp_ref, b2_ref, wl_ref, bl_ref, z_ref):
    dis = _dis_from(degp_ref)
    a = jnp.concatenate([acch_ref[0], acch_ref[1]], axis=1)
    h = jnp.maximum(a * dis + b2_ref[...], 0.0)
    z_ref[...] = jnp.dot(h, wl_ref[...],
                         preferred_element_type=jnp.float32) + bl_ref[...]


def _full(shape):
    return pl.BlockSpec(shape, lambda i: (0,) * len(shape))


_degp_spec = pl.BlockSpec((NC, _RB, 16), lambda i: (0, i, 0))
_acch_spec = pl.BlockSpec((NC, _RB, HH), lambda i: (0, i, 0))
_row_spec = pl.BlockSpec((_RB, H), lambda i: (i, 0))

_y1 = pl.pallas_call(
    _y1_body,
    grid=(N // _RB,),
    in_specs=[_degp_spec, _row_spec, _full((H, H))],
    out_specs=_acch_spec,
    out_shape=jax.ShapeDtypeStruct((NC, N, HH), jnp.float32),
)

_y2 = pl.pallas_call(
    _y2_body,
    grid=(N // _RB,),
    in_specs=[_acch_spec, _degp_spec, _full((1, H)), _full((H, H))],
    out_specs=_acch_spec,
    out_shape=jax.ShapeDtypeStruct((NC, N, HH), jnp.float32),
)

_z = pl.pallas_call(
    _z_body,
    grid=(N // _RB,),
    in_specs=[_acch_spec, _degp_spec, _full((1, H)), _full((H, 64)),
              _full((1, 64))],
    out_specs=pl.BlockSpec((_RB, 64), lambda i: (i, 0)),
    out_shape=jax.ShapeDtypeStruct((N, 64), jnp.float32),
)


def kernel(x, edge_index, W1, b1, W2, b2, Wl, bl):
    src = edge_index[0].reshape(E // CH, CH)
    dst = edge_index[1].reshape(E // CH, CH)
    degp = _deg_partials(dst)
    y1 = _y1(degp, x, W1)
    a1 = _agg_halves(y1, src, dst)
    y2 = _y2(a1, degp, b1.reshape(1, H), W2)
    a2 = _agg_halves(y2, src, dst)
    return _z(a2, degp, b2.reshape(1, H), Wl, bl.reshape(1, 64))


# speedup vs baseline: 33.0422x; 1.0001x over previous
import jax
import jax.numpy as jnp
from jax import lax
from jax.experimental import pallas as pl
from jax.experimental.pallas import tpu as pltpu
from jax.experimental.pallas import tpu_sc as plsc

N = 10000
E = 320000
H = 128
HH = H // 2
NC = 2
NS = 16
NW = NC * NS
CH = 125
EPW = E // NW
NCHUNK = EPW // CH
EPT = E // NS
NCHUNK2 = EPT // CH

RSTRIDE = 624
RLEN = 640

_mesh = plsc.VectorSubcoreMesh(core_axis_name="c", subcore_axis_name="s")


def _fill(buf, nrows, ncols, value):
    vec = jnp.full((16,), value, dtype=jnp.float32)

    def body(i, _):
        for j in range(ncols // 16):
            buf[i, pl.ds(j * 16, 16)] = vec
        return 0

    lax.fori_loop(0, nrows, body, 0)


def _deg_body(dst_hbm, out_hbm, dacc, dst_buf, ones, initbuf, gsem):
    c = lax.axis_index("c")
    s = lax.axis_index("s")
    wid = s * NC + c
    base = pl.multiple_of(s * RSTRIDE, 8)

    initval = jnp.where(c == 0, 1.0, 0.0)
    _fill(initbuf, RLEN, 16, initval)
    _fill(ones, CH, 16, 1.0)
    pltpu.sync_copy(dst_hbm.at[pl.ds(wid * NCHUNK, NCHUNK)], dst_buf)
    pltpu.sync_copy(initbuf, dacc.at[pl.ds(base, RLEN)])
    plsc.subcore_barrier()

    def body(k, _):
        for j in range(8):
            pltpu.async_copy(ones, dacc.at[dst_buf.at[8 * k + j]], gsem,
                             add=True)
        for j in range(8):
            pltpu.make_async_copy(ones, dacc.at[dst_buf.at[0]], gsem).wait()
        return 0

    lax.fori_loop(0, NCHUNK // 8, body, 0)
    plsc.subcore_barrier()
    pltpu.sync_copy(dacc.at[pl.ds(base, RLEN)], out_hbm.at[c, pl.ds(base, RLEN)])


_deg_partials = pl.kernel(
    _deg_body,
    out_type=jax.ShapeDtypeStruct((NC, N, 16), jnp.float32),
    mesh=_mesh,
    scratch_types=[
        pltpu.VMEM_SHARED((N, 16), jnp.float32),
        pltpu.VMEM((NCHUNK, CH), jnp.int32),
        pltpu.VMEM((CH, 16), jnp.float32),
        pltpu.VMEM((RLEN, 16), jnp.float32),
        pltpu.SemaphoreType.DMA,
    ],
    compiler_params=pltpu.CompilerParams(use_tc_tiling_on_sc=False),
)


NBUF = 5
ADV = 3


def _agg_body(y_hbm, src_hbm, dst_hbm, out_hbm, acc, src_buf, dst_buf, rows,
              *sems):
    c = lax.axis_index("c")
    s = lax.axis_index("s")
    base = pl.multiple_of(s * RSTRIDE, 8)
    row0 = s * NCHUNK2
    gsem = sems[:NBUF]
    ssem = sems[NBUF:]

    pltpu.sync_copy(src_hbm.at[pl.ds(row0, NCHUNK2)], src_buf)
    pltpu.sync_copy(dst_hbm.at[pl.ds(row0, NCHUNK2)], dst_buf)
    pltpu.sync_copy(y_hbm.at[c, pl.ds(base, RLEN)], acc.at[pl.ds(base, RLEN)])
    plsc.subcore_barrier()

    yc = y_hbm.at[c]

    for p in range(ADV):
        pltpu.async_copy(yc.at[src_buf.at[p]], rows.at[p], gsem[p])

    def outer(k, _):
        for b in range(NBUF):
            chunk = NBUF * k + b
            nb = (b + ADV) % NBUF

            @pl.when(chunk + ADV < NCHUNK2)
            def _():
                @pl.when(chunk + ADV >= NBUF)
                def _():
                    pltpu.make_async_copy(rows.at[nb], acc.at[dst_buf.at[0]],
                                          ssem[nb]).wait()
                pltpu.async_copy(yc.at[src_buf.at[chunk + ADV]], rows.at[nb],
                                 gsem[nb])

            pltpu.make_async_copy(yc.at[src_buf.at[chunk]], rows.at[b],
                                  gsem[b]).wait()
            pltpu.async_copy(rows.at[b], acc.at[dst_buf.at[chunk]], ssem[b],
                             add=True)
        return 0

    lax.fori_loop(0, NCHUNK2 // NBUF, outer, 0)
    for b in range(NBUF):
        pltpu.make_async_copy(rows.at[b], acc.at[dst_buf.at[0]],
                              ssem[b]).wait()
    plsc.subcore_barrier()
    pltpu.sync_copy(acc.at[pl.ds(base, RLEN)], out_hbm.at[c, pl.ds(base, RLEN)])


_agg_halves = pl.kernel(
    _agg_body,
    out_type=jax.ShapeDtypeStruct((NC, N, HH), jnp.float32),
    mesh=_mesh,
    scratch_types=[
        pltpu.VMEM_SHARED((N, HH), jnp.float32),
        pltpu.VMEM((NCHUNK2, CH), jnp.int32),
        pltpu.VMEM((NCHUNK2, CH), jnp.int32),
        pltpu.VMEM((NBUF, CH, HH), jnp.float32),
    ] + [pltpu.SemaphoreType.DMA] * (2 * NBUF),
    compiler_params=pltpu.CompilerParams(use_tc_tiling_on_sc=False),
)


_RB = 5000


def _dis_from(degp):
    deg = degp[0] + degp[1]
    return lax.rsqrt(jnp.maximum(deg[:, 0:1], 1.0))


def _xw_body(x_ref, w_ref, o_ref):
    o_ref[...] = jnp.dot(x_ref[...], w_ref[...],
                         preferred_element_type=jnp.float32)


def _scale_body(degp_ref, xw_ref, y_ref):
    dis = _dis_from(degp_ref)
    y = xw_ref[...] * dis
    y_ref[0] = y[:, :HH]
    y_ref[1] = y[:, HH:]


def _y2_body(acch_ref, degp_ref, b_ref, w_ref, y_ref):
    dis = _dis_from(degp_ref)
    a = jnp.concatenate([acch_ref[0], acch_ref[1]], axis=1)
    h = jnp.maximum(a * dis + b_ref[...], 0.0)
    y = jnp.dot(h, w_ref[...], preferred_element_type=jnp.float32) * dis
    y_ref[0] = y[:, :HH]
    y_ref[1] = y[:, HH:]


def _z_body(acch_ref, degp_ref, b2_ref, wl_ref, bl_ref, z_ref):
    dis = _dis_from(degp_ref)
    a = jnp.concatenate([acch_ref[0], acch_ref[1]], axis=1)
    h = jnp.maximum(a * dis + b2_ref[...], 0.0)
    z_ref[...] = jnp.dot(h, wl_ref[...],
                         preferred_element_type=jnp.float32) + bl_ref[...]


def _full(shape):
    return pl.BlockSpec(shape, lambda i: (0,) * len(shape))


_degp_spec = pl.BlockSpec((NC, _RB, 16), lambda i: (0, i, 0))
_acch_spec = pl.BlockSpec((NC, _RB, HH), lambda i: (0, i, 0))
_row_spec = pl.BlockSpec((_RB, H), lambda i: (i, 0))

_xw = pl.pallas_call(
    _xw_body,
    grid=(N // _RB,),
    in_specs=[_row_spec, _full((H, H))],
    out_specs=_row_spec,
    out_shape=jax.ShapeDtypeStruct((N, H), jnp.float32),
)

_scale = pl.pallas_call(
    _scale_body,
    grid=(N // _RB,),
    in_specs=[_degp_spec, _row_spec],
    out_specs=_acch_spec,
    out_shape=jax.ShapeDtypeStruct((NC, N, HH), jnp.float32),
)

_y2 = pl.pallas_call(
    _y2_body,
    grid=(N // _RB,),
    in_specs=[_acch_spec, _degp_spec, _full((1, H)), _full((H, H))],
    out_specs=_acch_spec,
    out_shape=jax.ShapeDtypeStruct((NC, N, HH), jnp.float32),
)

_z = pl.pallas_call(
    _z_body,
    grid=(N // _RB,),
    in_specs=[_acch_spec, _degp_spec, _full((1, H)), _full((H, 64)),
              _full((1, 64))],
    out_specs=pl.BlockSpec((_RB, 64), lambda i: (i, 0)),
    out_shape=jax.ShapeDtypeStruct((N, 64), jnp.float32),
)


def kernel(x, edge_index, W1, b1, W2, b2, Wl, bl):
    src = edge_index[0].reshape(E // CH, CH)
    dst = edge_index[1].reshape(E // CH, CH)
    degp = _deg_partials(dst)
    xw = _xw(x, W1)
    y1 = _scale(degp, xw)
    a1 = _agg_halves(y1, src, dst)
    y2 = _y2(a1, degp, b1.reshape(1, H), W2)
    a2 = _agg_halves(y2, src, dst)
    return _z(a2, degp, b2.reshape(1, H), Wl, bl.reshape(1, 64))
